# baseline (speedup 1.0000x reference)
import jax
import jax.numpy as jnp
from jax import lax
from jax.experimental import pallas as pl
from jax.experimental.pallas import tpu as pltpu
from jax.experimental.pallas import tpu_sc as plsc

D = 128
NC = 2
NS = 16
NW = NC * NS
SEQ_PAD = 56
UNROLL = 4


def _permute_body(x_hbm, perm_hbm, out_hbm,
                  perm_v, in0, in1, out0, out1, si0, si1, so0, so1):
    batch, seq, _ = x_hbm.shape
    b_per_w = batch // NW
    half = b_per_w // 2
    wid = lax.axis_index("s") * NC + lax.axis_index("c")
    base = wid * b_per_w

    pltpu.sync_copy(perm_hbm, perm_v)
    pvecs = [perm_v[pl.ds(16 * j, 16)] for j in range(D // 16)]

    def compute(in_v, out_v):
        @plsc.parallel_loop(0, seq, unroll=UNROLL)
        def _(r):
            rs = jnp.full((16,), r, jnp.int32)
            for j in range(D // 16):
                v = plsc.load_gather(in_v, [rs, pvecs[j]])
                out_v[r, pl.ds(16 * j, 16)] = v

    def copy_in(c, in_v, si):
        return pltpu.make_async_copy(
            x_hbm.at[base + c], in_v.at[pl.ds(0, seq)], si)

    def copy_out(c, out_v, so):
        return pltpu.make_async_copy(
            out_v.at[pl.ds(0, seq)], out_hbm.at[base + c], so)

    copy_in(0, in0, si0).start()
    copy_in(1, in1, si1).start()

    def loop_body(ci2, carry):
        for par, (in_v, out_v, si, so) in enumerate(
                ((in0, out0, si0, so0), (in1, out1, si1, so1))):
            c = 2 * ci2 + par
            copy_in(c, in_v, si).wait()

            @pl.when(ci2 > 0)
            def _():
                copy_out(c - 2, out_v, so).wait()

            compute(in_v, out_v)
            copy_out(c, out_v, so).start()

            @pl.when(ci2 < half - 1)
            def _():
                copy_in(c + 2, in_v, si).start()
        return carry

    lax.fori_loop(0, half, loop_body, 0)
    copy_out(2 * half - 2, out0, so0).wait()
    copy_out(2 * half - 1, out1, so1).wait()


def kernel(x, permutation):
    b, s, d = x.shape
    perm = permutation.astype(jnp.int32)

    mesh = plsc.VectorSubcoreMesh(core_axis_name="c", subcore_axis_name="s")
    run = pl.kernel(
        _permute_body,
        out_type=jax.ShapeDtypeStruct((b, s, d), jnp.float32),
        mesh=mesh,
        scratch_types=[
            pltpu.VMEM((D,), jnp.int32),
            pltpu.VMEM((SEQ_PAD, D), jnp.float32),
            pltpu.VMEM((SEQ_PAD, D), jnp.float32),
            pltpu.VMEM((SEQ_PAD, D), jnp.float32),
            pltpu.VMEM((SEQ_PAD, D), jnp.float32),
            pltpu.SemaphoreType.DMA,
            pltpu.SemaphoreType.DMA,
            pltpu.SemaphoreType.DMA,
            pltpu.SemaphoreType.DMA,
        ],
        compiler_params=pltpu.CompilerParams(
            needs_layout_passes=False, use_tc_tiling_on_sc=True),
    )
    return run(x, perm)


# speedup vs baseline: 2.6950x; 2.6950x over previous
import jax
import jax.numpy as jnp
from jax import lax
from jax.experimental import pallas as pl
from jax.experimental.pallas import tpu as pltpu
from jax.experimental.pallas import tpu_sc as plsc

D = 128
NC = 2
NS = 16
NW = NC * NS
CHUNK = 128
UNROLL = 4


def _permute_body(x_hbm, perm_hbm, out_hbm,
                  perm_v, in0, in1, out0, out1, si0, si1, so0, so1):
    rows = x_hbm.shape[0]
    rows_per_w = rows // NW
    half = rows_per_w // CHUNK // 2
    wid = lax.axis_index("s") * NC + lax.axis_index("c")
    base = wid * rows_per_w

    pltpu.sync_copy(perm_hbm, perm_v)
    pvecs = [perm_v[pl.ds(16 * j, 16)] for j in range(D // 16)]

    def compute(in_v, out_v):
        @plsc.parallel_loop(0, CHUNK, unroll=UNROLL)
        def _(r):
            rs = jnp.full((16,), r, jnp.int32)
            for j in range(D // 16):
                v = plsc.load_gather(in_v, [rs, pvecs[j]])
                out_v[r, pl.ds(16 * j, 16)] = v

    def copy_in(c, in_v, si):
        return pltpu.make_async_copy(
            x_hbm.at[pl.ds(base + c * CHUNK, CHUNK)], in_v, si)

    def copy_out(c, out_v, so):
        return pltpu.make_async_copy(
            out_v, out_hbm.at[pl.ds(base + c * CHUNK, CHUNK)], so)

    copy_in(0, in0, si0).start()
    copy_in(1, in1, si1).start()

    def loop_body(ci2, carry):
        for par, (in_v, out_v, si, so) in enumerate(
                ((in0, out0, si0, so0), (in1, out1, si1, so1))):
            c = 2 * ci2 + par
            copy_in(c, in_v, si).wait()

            @pl.when(ci2 > 0)
            def _():
                copy_out(c - 2, out_v, so).wait()

            compute(in_v, out_v)
            copy_out(c, out_v, so).start()

            @pl.when(ci2 < half - 1)
            def _():
                copy_in(c + 2, in_v, si).start()
        return carry

    lax.fori_loop(0, half, loop_body, 0)
    copy_out(2 * half - 2, out0, so0).wait()
    copy_out(2 * half - 1, out1, so1).wait()


def kernel(x, permutation):
    b, s, d = x.shape
    rows = b * s
    xt = jnp.transpose(x, (1, 0, 2)).reshape(rows, d)
    perm = permutation.astype(jnp.int32)

    mesh = plsc.VectorSubcoreMesh(core_axis_name="c", subcore_axis_name="s")
    run = pl.kernel(
        _permute_body,
        out_type=jax.ShapeDtypeStruct((rows, d), jnp.float32),
        mesh=mesh,
        scratch_types=[
            pltpu.VMEM((D,), jnp.int32),
            pltpu.VMEM((CHUNK, D), jnp.float32),
            pltpu.VMEM((CHUNK, D), jnp.float32),
            pltpu.VMEM((CHUNK, D), jnp.float32),
            pltpu.VMEM((CHUNK, D), jnp.float32),
            pltpu.SemaphoreType.DMA,
            pltpu.SemaphoreType.DMA,
            pltpu.SemaphoreType.DMA,
            pltpu.SemaphoreType.DMA,
        ],
        compiler_params=pltpu.CompilerParams(
            needs_layout_passes=False, use_tc_tiling_on_sc=True),
    )
    out = run(xt, perm)
    return jnp.transpose(out.reshape(s, b, d), (1, 0, 2))
